# baseline (speedup 1.0000x reference)
import jax
import jax.numpy as jnp
from jax.experimental import pallas as pl
from jax.experimental.pallas import tpu as pltpu

_EPS = 1e-5


def _bdot(a, b):
    return jnp.dot(a.astype(jnp.bfloat16), b.astype(jnp.bfloat16),
                   preferred_element_type=jnp.float32)


def _layer1_body(sup_ref, x_ref, w1_ref, b1_ref, gamma_ref, beta_ref, w2_ref,
                 supout_ref, bmat_ref, a_scr, r1_scr, stats_scr, *, bm, g, n):
    i = pl.program_id(0)

    @pl.when(i == 0)
    def _prologue():
        a_scr[...] = _bdot(x_ref[...], w1_ref[...]).astype(jnp.bfloat16)
        stats_scr[...] = jnp.zeros_like(stats_scr)

    @pl.when(i < g)
    def _stream():
        sup = sup_ref[...]
        r = jnp.maximum(_bdot(sup, a_scr[...]) + b1_ref[...], 0.0)
        r1_scr[pl.ds(i * bm, bm), :] = r
        stats_scr[0, :] += jnp.sum(r, axis=0)
        stats_scr[1, :] += jnp.sum(r * r, axis=0)
        supout_ref[...] = sup

    @pl.when(i == g)
    def _epilogue():
        mu = stats_scr[0, :] / n
        var = stats_scr[1, :] / n - mu * mu
        scale = gamma_ref[0, :] / jnp.sqrt(var + _EPS)
        shift = beta_ref[0, :] - mu * scale
        h = r1_scr[...] * scale[None, :] + shift[None, :]
        bmat_ref[...] = _bdot(h, w2_ref[...]).astype(jnp.bfloat16)


def _layer2_body(sup_ref, bmat_ref, b2_ref, gamma_ref, beta_ref,
                 out_ref, r2_scr, stats_scr, *, bm, g, n):
    i = pl.program_id(0)

    @pl.when(i == 0)
    def _init():
        stats_scr[...] = jnp.zeros_like(stats_scr)

    @pl.when(i < g)
    def _stream():
        r = jnp.maximum(_bdot(sup_ref[...], bmat_ref[...]) + b2_ref[...], 0.0)
        r2_scr[pl.ds(i * bm, bm), :] = r
        stats_scr[0, :] += jnp.sum(r, axis=0)
        stats_scr[1, :] += jnp.sum(r * r, axis=0)

    @pl.when(i == g)
    def _epilogue():
        mu = stats_scr[0, :] / n
        var = stats_scr[1, :] / n - mu * mu
        scale = gamma_ref[0, :] / jnp.sqrt(var + _EPS)
        shift = beta_ref[0, :] - mu * scale
        out_ref[...] = r2_scr[...] * scale[None, :] + shift[None, :]


def _pick_block(n, target):
    best = 8
    for cand in range(8, min(n, target) + 1, 8):
        if n % cand == 0:
            best = cand
    return best


def kernel(x, support, W1, b1, gamma1, beta1, W2, b2, gamma2, beta2):
    import functools
    n = support.shape[0]
    d_in = W1.shape[0]
    d_h = W1.shape[1]
    d_out = W2.shape[1]

    bm1 = _pick_block(n, 80)
    g1 = n // bm1
    sup_blk1 = pl.BlockSpec((bm1, n), lambda i: (min(i, g1 - 1) if isinstance(i, int) else jnp.minimum(i, g1 - 1), 0))
    supout, bmat = pl.pallas_call(
        functools.partial(_layer1_body, bm=bm1, g=g1, n=float(n)),
        grid=(g1 + 1,),
        in_specs=[
            sup_blk1,
            pl.BlockSpec((n, d_in), lambda i: (0, 0)),
            pl.BlockSpec((d_in, d_h), lambda i: (0, 0)),
            pl.BlockSpec((1, d_h), lambda i: (0, 0)),
            pl.BlockSpec((1, d_h), lambda i: (0, 0)),
            pl.BlockSpec((1, d_h), lambda i: (0, 0)),
            pl.BlockSpec((d_h, d_out), lambda i: (0, 0)),
        ],
        out_specs=[
            sup_blk1,
            pl.BlockSpec((n, d_out), lambda i: (0, 0)),
        ],
        out_shape=[
            jax.ShapeDtypeStruct((n, n), jnp.float32),
            jax.ShapeDtypeStruct((n, d_out), jnp.bfloat16),
        ],
        scratch_shapes=[
            pltpu.VMEM((n, d_h), jnp.bfloat16),
            pltpu.VMEM((n, d_h), jnp.float32),
            pltpu.VMEM((2, d_h), jnp.float32),
        ],
        compiler_params=pltpu.CompilerParams(
            dimension_semantics=("arbitrary",)),
    )(support, x, W1, b1.reshape(1, d_h), gamma1.reshape(1, d_h),
      beta1.reshape(1, d_h), W2)

    bm2 = _pick_block(n, 400)
    g2 = n // bm2
    sup_blk2 = pl.BlockSpec((bm2, n), lambda i: (min(i, g2 - 1) if isinstance(i, int) else jnp.minimum(i, g2 - 1), 0))
    out = pl.pallas_call(
        functools.partial(_layer2_body, bm=bm2, g=g2, n=float(n)),
        grid=(g2 + 1,),
        in_specs=[
            sup_blk2,
            pl.BlockSpec((n, d_out), lambda i: (0, 0)),
            pl.BlockSpec((1, d_out), lambda i: (0, 0)),
            pl.BlockSpec((1, d_out), lambda i: (0, 0)),
            pl.BlockSpec((1, d_out), lambda i: (0, 0)),
        ],
        out_specs=pl.BlockSpec((n, d_out), lambda i: (0, 0)),
        out_shape=jax.ShapeDtypeStruct((n, d_out), jnp.float32),
        scratch_shapes=[
            pltpu.VMEM((n, d_out), jnp.float32),
            pltpu.VMEM((2, d_out), jnp.float32),
        ],
        compiler_params=pltpu.CompilerParams(
            dimension_semantics=("arbitrary",)),
    )(support, bmat, b2.reshape(1, d_out), gamma2.reshape(1, d_out),
      beta2.reshape(1, d_out))

    return (out, supout)


# speedup vs baseline: 1.0638x; 1.0638x over previous
import jax
import jax.numpy as jnp
from jax.experimental import pallas as pl
from jax.experimental.pallas import tpu as pltpu

_EPS = 1e-5


def _bdot(a, b):
    return jnp.dot(a.astype(jnp.bfloat16), b.astype(jnp.bfloat16),
                   preferred_element_type=jnp.float32)


def _layer1_body(sup_ref, x_ref, w1_ref, b1_ref, gamma_ref, beta_ref, w2_ref,
                 supout_ref, bmat_ref, a_scr, r1_scr, stats_scr, *, bm, g, n):
    i = pl.program_id(0)

    @pl.when(i == 0)
    def _prologue():
        a_scr[...] = _bdot(x_ref[...], w1_ref[...]).astype(jnp.bfloat16)
        stats_scr[...] = jnp.zeros_like(stats_scr)

    @pl.when(i < g)
    def _stream():
        sup = sup_ref[...]
        r = jnp.maximum(_bdot(sup, a_scr[...]) + b1_ref[...], 0.0)
        r1_scr[pl.ds(i * bm, bm), :] = r
        stats_scr[0, :] += jnp.sum(r, axis=0)
        stats_scr[1, :] += jnp.sum(r * r, axis=0)
        supout_ref[...] = sup

    @pl.when(i == g)
    def _epilogue():
        mu = stats_scr[0, :] / n
        var = stats_scr[1, :] / n - mu * mu
        scale = gamma_ref[0, :] / jnp.sqrt(var + _EPS)
        shift = beta_ref[0, :] - mu * scale
        h = r1_scr[...] * scale[None, :] + shift[None, :]
        bmat_ref[...] = _bdot(h, w2_ref[...]).astype(jnp.bfloat16)


def _layer2_body(sup_ref, bmat_ref, b2_ref, gamma_ref, beta_ref,
                 out_ref, r2_scr, stats_scr, *, bm, g, n):
    i = pl.program_id(0)

    @pl.when(i == 0)
    def _init():
        stats_scr[...] = jnp.zeros_like(stats_scr)

    @pl.when(i < g)
    def _stream():
        r = jnp.maximum(_bdot(sup_ref[...], bmat_ref[...]) + b2_ref[...], 0.0)
        r2_scr[pl.ds(i * bm, bm), :] = r
        stats_scr[0, :] += jnp.sum(r, axis=0)
        stats_scr[1, :] += jnp.sum(r * r, axis=0)

    @pl.when(i == g)
    def _epilogue():
        mu = stats_scr[0, :] / n
        var = stats_scr[1, :] / n - mu * mu
        scale = gamma_ref[0, :] / jnp.sqrt(var + _EPS)
        shift = beta_ref[0, :] - mu * scale
        out_ref[...] = r2_scr[...] * scale[None, :] + shift[None, :]


def _pick_block(n, target):
    best = 8
    for cand in range(8, min(n, target) + 1, 8):
        if n % cand == 0:
            best = cand
    return best


def kernel(x, support, W1, b1, gamma1, beta1, W2, b2, gamma2, beta2):
    import functools
    n = support.shape[0]
    d_in = W1.shape[0]
    d_h = W1.shape[1]
    d_out = W2.shape[1]

    bm1 = _pick_block(n, 200)
    g1 = n // bm1
    sup_blk1 = pl.BlockSpec((bm1, n), lambda i: (min(i, g1 - 1) if isinstance(i, int) else jnp.minimum(i, g1 - 1), 0))
    supout, bmat = pl.pallas_call(
        functools.partial(_layer1_body, bm=bm1, g=g1, n=float(n)),
        grid=(g1 + 1,),
        in_specs=[
            sup_blk1,
            pl.BlockSpec((n, d_in), lambda i: (0, 0)),
            pl.BlockSpec((d_in, d_h), lambda i: (0, 0)),
            pl.BlockSpec((1, d_h), lambda i: (0, 0)),
            pl.BlockSpec((1, d_h), lambda i: (0, 0)),
            pl.BlockSpec((1, d_h), lambda i: (0, 0)),
            pl.BlockSpec((d_h, d_out), lambda i: (0, 0)),
        ],
        out_specs=[
            sup_blk1,
            pl.BlockSpec((n, d_out), lambda i: (0, 0)),
        ],
        out_shape=[
            jax.ShapeDtypeStruct((n, n), jnp.float32),
            jax.ShapeDtypeStruct((n, d_out), jnp.bfloat16),
        ],
        scratch_shapes=[
            pltpu.VMEM((n, d_h), jnp.bfloat16),
            pltpu.VMEM((n, d_h), jnp.float32),
            pltpu.VMEM((2, d_h), jnp.float32),
        ],
        compiler_params=pltpu.CompilerParams(
            dimension_semantics=("arbitrary",)),
    )(support, x, W1, b1.reshape(1, d_h), gamma1.reshape(1, d_h),
      beta1.reshape(1, d_h), W2)

    bm2 = _pick_block(n, 400)
    g2 = n // bm2
    sup_blk2 = pl.BlockSpec((bm2, n), lambda i: (min(i, g2 - 1) if isinstance(i, int) else jnp.minimum(i, g2 - 1), 0))
    out = pl.pallas_call(
        functools.partial(_layer2_body, bm=bm2, g=g2, n=float(n)),
        grid=(g2 + 1,),
        in_specs=[
            sup_blk2,
            pl.BlockSpec((n, d_out), lambda i: (0, 0)),
            pl.BlockSpec((1, d_out), lambda i: (0, 0)),
            pl.BlockSpec((1, d_out), lambda i: (0, 0)),
            pl.BlockSpec((1, d_out), lambda i: (0, 0)),
        ],
        out_specs=pl.BlockSpec((n, d_out), lambda i: (0, 0)),
        out_shape=jax.ShapeDtypeStruct((n, d_out), jnp.float32),
        scratch_shapes=[
            pltpu.VMEM((n, d_out), jnp.float32),
            pltpu.VMEM((2, d_out), jnp.float32),
        ],
        compiler_params=pltpu.CompilerParams(
            dimension_semantics=("arbitrary",)),
    )(support, bmat, b2.reshape(1, d_out), gamma2.reshape(1, d_out),
      beta2.reshape(1, d_out))

    return (out, supout)
